# Initial kernel scaffold; baseline (speedup 1.0000x reference)
import jax
import jax.numpy as jnp
from jax.experimental import pallas as pl


def kernel(tokens, start_idxs, end_idxs, vocab_map):
    raise NotImplementedError("write your pallas kernel here")


# speedup vs baseline: 135.4163x; 135.4163x over previous
import functools

import jax
import jax.numpy as jnp
from jax import lax
from jax.experimental import pallas as pl
from jax.experimental.pallas import tpu as pltpu
from jax.experimental.pallas import tpu_sc as plsc

_LANES = 16
_NUM_WORKERS = 32
_CHUNK = 12800


@functools.partial(jax.jit, static_argnums=(2,))
def _sc_lookup(vocab_map, flat_tokens, n_per_worker):
    n_chunks = n_per_worker // _CHUNK
    mesh = plsc.VectorSubcoreMesh(
        core_axis_name="c", subcore_axis_name="s", num_cores=2, num_subcores=16
    )

    @functools.partial(
        pl.kernel,
        out_type=jax.ShapeDtypeStruct(flat_tokens.shape, jnp.float32),
        mesh=mesh,
        scratch_types=[
            pltpu.VMEM(vocab_map.shape, jnp.float32),
            pltpu.VMEM((_CHUNK,), jnp.int32),
            pltpu.VMEM((_CHUNK,), jnp.float32),
        ],
        compiler_params=pltpu.CompilerParams(
            use_tc_tiling_on_sc=False, needs_layout_passes=False
        ),
    )
    def body(table_hbm, tok_hbm, out_hbm, table_v, idx_v, out_v):
        wid = lax.axis_index("s") * 2 + lax.axis_index("c")
        base = wid * n_per_worker
        pltpu.sync_copy(table_hbm, table_v)
        for c in range(n_chunks):
            off = base + c * _CHUNK
            pltpu.sync_copy(tok_hbm.at[pl.ds(off, _CHUNK)], idx_v)

            def step(i, carry):
                sl = pl.ds(i * _LANES, _LANES)
                out_v[sl] = plsc.load_gather(table_v, [idx_v[sl]])
                return carry

            lax.fori_loop(0, _CHUNK // _LANES, step, 0)
            pltpu.sync_copy(out_v, out_hbm.at[pl.ds(off, _CHUNK)])

    return body(vocab_map, flat_tokens)


def kernel(tokens, start_idxs, end_idxs, vocab_map):
    b, s = tokens.shape
    n = b * s
    token_ids = _sc_lookup(vocab_map, tokens.reshape(n), n // _NUM_WORKERS)
    return token_ids.reshape(b, s), start_idxs, end_idxs


# speedup vs baseline: 156.2552x; 1.1539x over previous
import functools

import jax
import jax.numpy as jnp
from jax import lax
from jax.experimental import pallas as pl
from jax.experimental.pallas import tpu as pltpu
from jax.experimental.pallas import tpu_sc as plsc

_LANES = 16
_NUM_WORKERS = 32
_CHUNK = 6400
_NBUF = 2


@functools.partial(jax.jit, static_argnums=(2,))
def _sc_lookup(vocab_map, flat_tokens, n_per_worker):
    n_chunks = n_per_worker // _CHUNK
    mesh = plsc.VectorSubcoreMesh(
        core_axis_name="c", subcore_axis_name="s", num_cores=2, num_subcores=16
    )

    @functools.partial(
        pl.kernel,
        out_type=jax.ShapeDtypeStruct(flat_tokens.shape, jnp.float32),
        mesh=mesh,
        scratch_types=[
            pltpu.VMEM(vocab_map.shape, jnp.float32),
            [pltpu.VMEM((_CHUNK,), jnp.int32) for _ in range(_NBUF)],
            [pltpu.VMEM((_CHUNK,), jnp.float32) for _ in range(_NBUF)],
            pltpu.SemaphoreType.DMA,
            [pltpu.SemaphoreType.DMA for _ in range(_NBUF)],
            [pltpu.SemaphoreType.DMA for _ in range(_NBUF)],
        ],
        compiler_params=pltpu.CompilerParams(
            use_tc_tiling_on_sc=False, needs_layout_passes=False
        ),
    )
    def body(table_hbm, tok_hbm, out_hbm, table_v, idx_v, out_v,
             sem_tab, sem_in, sem_out):
        wid = lax.axis_index("s") * 2 + lax.axis_index("c")
        base = wid * n_per_worker

        cp_tab = pltpu.async_copy(table_hbm, table_v, sem_tab)
        in_cps = [None] * _NBUF
        out_cps = [None] * _NBUF
        for c in range(min(_NBUF, n_chunks)):
            in_cps[c] = pltpu.async_copy(
                tok_hbm.at[pl.ds(base + c * _CHUNK, _CHUNK)],
                idx_v[c], sem_in[c],
            )
        cp_tab.wait()

        for c in range(n_chunks):
            b = c % _NBUF
            in_cps[b].wait()
            if out_cps[b] is not None:
                out_cps[b].wait()

            @plsc.parallel_loop(0, _CHUNK, step=_LANES, unroll=8)
            def _(i):
                out_v[b][pl.ds(i, _LANES)] = plsc.load_gather(
                    table_v, [idx_v[b][pl.ds(i, _LANES)]]
                )

            out_cps[b] = pltpu.async_copy(
                out_v[b], out_hbm.at[pl.ds(base + c * _CHUNK, _CHUNK)],
                sem_out[b],
            )
            nxt = c + _NBUF
            if nxt < n_chunks:
                in_cps[b] = pltpu.async_copy(
                    tok_hbm.at[pl.ds(base + nxt * _CHUNK, _CHUNK)],
                    idx_v[b], sem_in[b],
                )
        for b in range(min(_NBUF, n_chunks)):
            if out_cps[b] is not None:
                out_cps[b].wait()

    return body(vocab_map, flat_tokens)


def kernel(tokens, start_idxs, end_idxs, vocab_map):
    b, s = tokens.shape
    n = b * s
    token_ids = _sc_lookup(vocab_map, tokens.reshape(n), n // _NUM_WORKERS)
    return token_ids.reshape(b, s), start_idxs, end_idxs
